# baseline (speedup 1.0000x reference)
import functools
import math

import jax
import jax.numpy as jnp
from jax import lax
from jax.experimental import pallas as pl
from jax.experimental.pallas import tpu as pltpu
from jax.experimental.pallas import tpu_sc as plsc

D_MODEL = 64
SCALE = math.sqrt(D_MODEL)
NUM_CORES = 2
NUM_SUBCORES = 16
NUM_WORKERS = NUM_CORES * NUM_SUBCORES
LANES = 16
NBUF = 4


def kernel(input_ids, embedding_weight):
    batch, seq = input_ids.shape
    rows_per_worker = batch // NUM_WORKERS
    assert batch % NUM_WORKERS == 0 and rows_per_worker >= 2 * NBUF

    mesh = plsc.VectorSubcoreMesh(
        core_axis_name="c", subcore_axis_name="s",
        num_cores=NUM_CORES, num_subcores=NUM_SUBCORES)

    @functools.partial(
        pl.kernel,
        mesh=mesh,
        out_type=jax.ShapeDtypeStruct((batch, seq, D_MODEL), jnp.float32),
        scratch_types=[
            pltpu.VMEM((rows_per_worker, seq), jnp.int32),
            pltpu.VMEM((NBUF, seq, D_MODEL), jnp.float32),
            [pltpu.SemaphoreType.DMA] * NBUF,
            [pltpu.SemaphoreType.DMA] * NBUF,
        ],
        compiler_params=pltpu.CompilerParams(use_tc_tiling_on_sc=False),
    )
    def emb(idx_hbm, table_hbm, out_hbm, idx_v, rows_v, sem_g, sem_w):
        wid = lax.axis_index("s") * NUM_CORES + lax.axis_index("c")
        base = wid * rows_per_worker

        pltpu.sync_copy(idx_hbm.at[pl.ds(base, rows_per_worker)], idx_v)

        def issue_gather(g, b):
            pltpu.async_copy(table_hbm.at[idx_v.at[g]], rows_v.at[b], sem_g[b])

        def wait_gather(g, b):
            pltpu.make_async_copy(
                table_hbm.at[idx_v.at[g]], rows_v.at[b], sem_g[b]).wait()

        def scale_rows(b):
            @plsc.parallel_loop(0, seq, unroll=4)
            def _(i):
                for j in range(D_MODEL // LANES):
                    sl = pl.ds(j * LANES, LANES)
                    rows_v[b, i, sl] = rows_v[b, i, sl] * SCALE

        def issue_write(g, b):
            pltpu.async_copy(rows_v.at[b], out_hbm.at[base + g], sem_w[b])

        def wait_write(g, b):
            pltpu.make_async_copy(
                rows_v.at[b], out_hbm.at[base + g], sem_w[b]).wait()

        n = rows_per_worker

        issue_gather(0, 0)
        issue_gather(1, 1)
        for g in (0, 1):
            wait_gather(g, g)
            scale_rows(g)
            issue_write(g, g)
            issue_gather(g + 2, g + 2)

        def outer(g0, carry):
            for db in range(NBUF):
                b = (2 + db) % NBUF
                gg = g0 + db
                wait_gather(gg, b)
                scale_rows(b)
                issue_write(gg, b)
                wait_write(gg - 2, (b + 2) % NBUF)
                issue_gather(gg + 2, (b + 2) % NBUF)
            return carry

        n_main = (n - 4) // NBUF
        lax.fori_loop(0, n_main, lambda s_, c: outer(2 + s_ * NBUF, c), 0)

        for gg in (n - 2, n - 1):
            b = gg % NBUF
            wait_gather(gg, b)
            scale_rows(b)
            issue_write(gg, b)

        for gg in range(n - 4, n):
            wait_write(gg, gg % NBUF)

    return emb(input_ids, embedding_weight)


# speedup vs baseline: 1.0329x; 1.0329x over previous
import functools
import math

import jax
import jax.numpy as jnp
from jax import lax
from jax.experimental import pallas as pl
from jax.experimental.pallas import tpu as pltpu
from jax.experimental.pallas import tpu_sc as plsc

D_MODEL = 64
PAD_W = 128
SCALE = math.sqrt(D_MODEL)
NUM_CORES = 2
NUM_SUBCORES = 16
NUM_WORKERS = NUM_CORES * NUM_SUBCORES
LANES = 16
BB = 128
NBUF = 4


def kernel(input_ids, embedding_weight):
    batch, seq = input_ids.shape
    vocab, d_model = embedding_weight.shape
    assert batch == NUM_WORKERS * BB and d_model == D_MODEL

    ids_flat = input_ids.reshape(batch * seq)
    table_fat = jnp.pad(embedding_weight, ((0, 0), (0, PAD_W - D_MODEL)))

    mesh = plsc.VectorSubcoreMesh(
        core_axis_name="c", subcore_axis_name="s",
        num_cores=NUM_CORES, num_subcores=NUM_SUBCORES)

    @functools.partial(
        pl.kernel,
        mesh=mesh,
        out_type=jax.ShapeDtypeStruct((seq, D_MODEL, batch), jnp.float32),
        scratch_types=[
            pltpu.VMEM((BB * seq,), jnp.int32),
            pltpu.VMEM((NBUF, BB), jnp.int32),
            pltpu.VMEM((NBUF, BB, PAD_W), jnp.float32),
            pltpu.VMEM((NBUF, D_MODEL, BB), jnp.float32),
            [pltpu.SemaphoreType.DMA] * NBUF,
            [pltpu.SemaphoreType.DMA] * NBUF,
        ],
        compiler_params=pltpu.CompilerParams(
            use_tc_tiling_on_sc=True, needs_layout_passes=False),
    )
    def emb(ids_hbm, table_hbm, out_hbm, ids_v, idx_v, rows_v, plane_v,
            sem_g, sem_w):
        wid = lax.axis_index("s") * NUM_CORES + lax.axis_index("c")
        col0 = wid * BB

        pltpu.sync_copy(ids_hbm.at[pl.ds(col0 * seq, BB * seq)], ids_v)

        lane = lax.iota(jnp.int32, LANES)

        def build_idx(g, b):
            for bb0 in range(0, BB, LANES):
                pos = (lane + bb0) * seq + g
                idx_v[b, pl.ds(bb0, LANES)] = plsc.load_gather(ids_v, [pos])

        def issue_gather(g, b):
            build_idx(g, b)
            pltpu.async_copy(table_hbm.at[idx_v.at[b]], rows_v.at[b],
                             sem_g[b])

        def wait_gather(b):
            pltpu.make_async_copy(table_hbm.at[idx_v.at[b]], rows_v.at[b],
                                  sem_g[b]).wait()

        def transpose_scale(b):
            @plsc.parallel_loop(0, D_MODEL, unroll=2)
            def _(c):
                for bb0 in range(0, BB, LANES):
                    v = plsc.load_gather(rows_v.at[b], [lane + bb0,
                                                        c + lane * 0])
                    plane_v[b, c, pl.ds(bb0, LANES)] = v * SCALE

        def issue_write(g, b):
            pltpu.async_copy(plane_v.at[b],
                             out_hbm.at[g, :, pl.ds(col0, BB)], sem_w[b])

        def wait_write(g, b):
            pltpu.make_async_copy(plane_v.at[b],
                                  out_hbm.at[g, :, pl.ds(col0, BB)],
                                  sem_w[b]).wait()

        n = seq

        issue_gather(0, 0)
        issue_gather(1, 1)
        for g in (0, 1):
            wait_gather(g)
            transpose_scale(g)
            issue_write(g, g)
            issue_gather(g + 2, g + 2)

        def outer(g0, carry):
            for db in range(NBUF):
                b = (2 + db) % NBUF
                gg = g0 + db
                wait_gather(b)
                transpose_scale(b)
                issue_write(gg, b)
                wait_write(gg - 2, (b + 2) % NBUF)
                issue_gather(gg + 2, (b + 2) % NBUF)
            return carry

        n_main = (n - 4) // NBUF
        lax.fori_loop(0, n_main, lambda s_, c: outer(2 + s_ * NBUF, c), 0)

        for gg in (n - 2, n - 1):
            b = gg % NBUF
            wait_gather(b)
            transpose_scale(b)
            issue_write(gg, b)

        for gg in range(n - 4, n):
            wait_write(gg, gg % NBUF)

    out_t = emb(ids_flat, table_fat)
    return out_t.transpose(2, 0, 1)


# speedup vs baseline: 1.1412x; 1.1048x over previous
import functools
import math

import jax
import jax.numpy as jnp
from jax import lax
from jax.experimental import pallas as pl
from jax.experimental.pallas import tpu as pltpu
from jax.experimental.pallas import tpu_sc as plsc

D_MODEL = 64
PAD_W = 128
SCALE = math.sqrt(D_MODEL)
NUM_CORES = 2
NUM_SUBCORES = 16
NUM_WORKERS = NUM_CORES * NUM_SUBCORES
LANES = 16
BB = 128
NBUF = 4


def kernel(input_ids, embedding_weight):
    batch, seq = input_ids.shape
    vocab, d_model = embedding_weight.shape
    assert batch == NUM_WORKERS * BB and d_model == D_MODEL

    ids_flat = input_ids.reshape(batch * seq)
    table_fat = jnp.pad(embedding_weight, ((0, 0), (0, PAD_W - D_MODEL)))

    mesh = plsc.VectorSubcoreMesh(
        core_axis_name="c", subcore_axis_name="s",
        num_cores=NUM_CORES, num_subcores=NUM_SUBCORES)

    @functools.partial(
        pl.kernel,
        mesh=mesh,
        out_type=jax.ShapeDtypeStruct((seq, D_MODEL, batch), jnp.float32),
        scratch_types=[
            pltpu.VMEM((BB * seq,), jnp.int32),
            pltpu.VMEM((NBUF, BB), jnp.int32),
            pltpu.VMEM((NBUF, BB, PAD_W), jnp.float32),
            pltpu.VMEM((NBUF, D_MODEL, BB), jnp.float32),
            [pltpu.SemaphoreType.DMA] * NBUF,
            [pltpu.SemaphoreType.DMA] * NBUF,
        ],
        compiler_params=pltpu.CompilerParams(
            use_tc_tiling_on_sc=True, needs_layout_passes=False),
    )
    def emb(ids_hbm, table_hbm, out_hbm, ids_v, idx_v, rows_v, plane_v,
            sem_g, sem_w):
        wid = lax.axis_index("s") * NUM_CORES + lax.axis_index("c")
        col0 = wid * BB

        pltpu.sync_copy(ids_hbm.at[pl.ds(col0 * seq, BB * seq)], ids_v)

        lane = lax.iota(jnp.int32, LANES)

        def build_idx(g, b):
            for bb0 in range(0, BB, LANES):
                pos = (lane + bb0) * seq + g
                idx_v[b, pl.ds(bb0, LANES)] = plsc.load_gather(ids_v, [pos])

        def issue_gather(g, b):
            build_idx(g, b)
            pltpu.async_copy(table_hbm.at[idx_v.at[b]], rows_v.at[b],
                             sem_g[b])

        def wait_gather(b):
            pltpu.make_async_copy(table_hbm.at[idx_v.at[b]], rows_v.at[b],
                                  sem_g[b]).wait()

        rot = [(lane + d) & (LANES - 1) for d in range(LANES)]
        cols = [c0 + lane for c0 in range(0, D_MODEL, LANES)]

        def transpose_scale(b):
            @plsc.parallel_loop(0, BB, LANES)
            def _(bb0):
                for d in range(LANES):
                    row_idx = bb0 + rot[d]
                    for ci, c0 in enumerate(range(0, D_MODEL, LANES)):
                        v = plsc.load_gather(rows_v.at[b],
                                             [row_idx, cols[ci]])
                        plsc.store_scatter(plane_v.at[b],
                                           [cols[ci], row_idx], v * SCALE)

        def issue_write(g, b):
            pltpu.async_copy(plane_v.at[b],
                             out_hbm.at[g, :, pl.ds(col0, BB)], sem_w[b])

        def wait_write(g, b):
            pltpu.make_async_copy(plane_v.at[b],
                                  out_hbm.at[g, :, pl.ds(col0, BB)],
                                  sem_w[b]).wait()

        n = seq

        issue_gather(0, 0)
        issue_gather(1, 1)

        def outer(g0, carry):
            for db in range(NBUF):
                b = db
                gg = g0 * NBUF + db
                wait_gather(b)
                transpose_scale(b)
                issue_write(gg, b)
                nb = (b + 2) % NBUF

                @pl.when(gg + 2 < n)
                def _():
                    @pl.when(gg >= 2)
                    def _():
                        wait_write(gg - 2, nb)
                    issue_gather(gg + 2, nb)
            return carry

        lax.fori_loop(0, n // NBUF, outer, 0)

        for gg in range(n - 4, n):
            wait_write(gg, gg % NBUF)

    out_t = emb(ids_flat, table_fat)
    return out_t.transpose(2, 0, 1)
